# baseline (speedup 1.0000x reference)
import functools

import jax
import jax.numpy as jnp
from jax import lax
from jax.experimental import pallas as pl
from jax.experimental.pallas import tpu as pltpu
from jax.experimental.pallas import tpu_sc as plsc

N = 10000
B = 2
E = 160000
NS = 16
CHUNK = 128
NCH = 80
IG = 8
NOG = NCH // IG
E_PAD = NS * NCH * CHUNK
RPT = 632
NPAD = NS * RPT
BLK = 1000


def _make_sc_agg(gather):
  mesh = plsc.VectorSubcoreMesh(core_axis_name="c", subcore_axis_name="s")

  @functools.partial(
      pl.kernel,
      mesh=mesh,
      out_type=jax.ShapeDtypeStruct((B, NPAD, 128), jnp.float32),
      scratch_types=[
          pltpu.VMEM((IG, CHUNK), jnp.int32),
          pltpu.VMEM((IG, CHUNK), jnp.int32),
          pltpu.VMEM((2, CHUNK, 128), jnp.float32),
          pltpu.VMEM_SHARED((NPAD, 128), jnp.float32),
          pltpu.SemaphoreType.DMA,
          pltpu.SemaphoreType.DMA,
          pltpu.SemaphoreType.DMA,
          pltpu.SemaphoreType.DMA,
      ],
  )
  def k(tbl, srcm, dstm, zeros, out, src_i, dst_i, rows_v, accum,
        gs0, gs1, ss0, ss1):
    c = lax.axis_index("c")
    s = lax.axis_index("s")
    gsems = [gs0, gs1]
    ssems = [ss0, ss1]
    if not gather:
      pltpu.sync_copy(tbl, rows_v.at[0])
    pltpu.sync_copy(zeros.at[pl.ds(s * RPT, RPT)],
                    accum.at[pl.ds(s * RPT, RPT)])
    plsc.subcore_barrier()

    def body(og, carry):
      pltpu.sync_copy(srcm.at[c, s, pl.ds(og * IG, IG)], src_i)
      pltpu.sync_copy(dstm.at[s, pl.ds(og * IG, IG)], dst_i)
      if gather:
        half = CHUNK // 2

        def gfire(j, slot):
          return [
              pltpu.async_copy(tbl.at[src_i.at[j, pl.ds(0, half)]],
                               rows_v.at[slot, pl.ds(0, half)], gsems[slot]),
              pltpu.async_copy(tbl.at[src_i.at[j, pl.ds(half, half)]],
                               rows_v.at[slot, pl.ds(half, half)],
                               gsems[slot]),
          ]

        gd = [None] * IG
        sd = [None] * IG
        gd[0] = gfire(0, 0)
        for j in range(IG):
          if j >= 1:
            sd[j - 1].wait()
          if j + 1 < IG:
            gd[j + 1] = gfire(j + 1, (j + 1) % 2)
          for d in gd[j]:
            d.wait()
          sd[j] = pltpu.async_copy(rows_v.at[j % 2], accum.at[dst_i.at[j]],
                                   ssems[j % 2], add=True)
        sd[IG - 1].wait()
      else:
        for j in range(IG):
          pltpu.sync_copy(rows_v.at[0], accum.at[dst_i.at[j]], add=True)
      return carry

    lax.fori_loop(0, NOG, body, 0)
    plsc.subcore_barrier()
    pltpu.sync_copy(accum.at[pl.ds(s * RPT, RPT)],
                    out.at[c, pl.ds(s * RPT, RPT)])

  return k


_sc_agg_128 = _make_sc_agg(True)
_sc_count = _make_sc_agg(False)


def _dinv(deg_ref):
  return lax.rsqrt(jnp.maximum(deg_ref[:, 0:1], 1.0))


def _prep_body(x_ref, dout_ref, o_ref):
  o_ref[0] = x_ref[0] * _dinv(dout_ref)


def _l0_body(a_ref, din_ref, dout_ref, w_ref, b_ref, h_ref, lo_ref, hi_ref):
  a = a_ref[0] * _dinv(din_ref)
  h = jnp.dot(a, w_ref[...], preferred_element_type=jnp.float32) + b_ref[...]
  h = jnp.maximum(h, 0.0)
  h_ref[0] = h
  hs = h * _dinv(dout_ref)
  lo_ref[0] = hs[:, :128]
  hi_ref[0] = hs[:, 128:]


def _l1_body(alo_ref, ahi_ref, din_ref, dout_ref, w1_ref, b1_ref, w2_ref,
             h_ref, g_ref):
  din = _dinv(din_ref)
  h = (jnp.dot(alo_ref[0] * din, w1_ref[:128, :],
               preferred_element_type=jnp.float32)
       + jnp.dot(ahi_ref[0] * din, w1_ref[128:, :],
                 preferred_element_type=jnp.float32)
       + b1_ref[...])
  h = jnp.maximum(h, 0.0)
  h_ref[0] = h
  g_ref[0] = jnp.dot(h * _dinv(dout_ref), w2_ref[...],
                     preferred_element_type=jnp.float32)


def _l2_body(a_ref, din_ref, b_ref, o_ref):
  o_ref[0] = a_ref[0] * _dinv(din_ref) + b_ref[...]


def _row_spec(width):
  return pl.BlockSpec((1, BLK, width), lambda b, i: (b, i, 0))


def _deg_spec():
  return pl.BlockSpec((BLK, 128), lambda b, i: (i, 0))


def _full_spec(shape):
  return pl.BlockSpec(shape, lambda b, i: tuple(0 for _ in shape))


def _grid():
  return (B, N // BLK)


def kernel(node_feature, edge_index, W0, b0, W1, b1, W2, b2):
  src = edge_index[0].astype(jnp.int32)
  dst = edge_index[1].astype(jnp.int32)
  src, dst = lax.sort([src, dst], num_keys=1)
  pad = E_PAD - E
  src_g = jnp.concatenate([src, jnp.zeros((pad,), jnp.int32)])
  dst_p = jnp.concatenate([dst, jnp.full((pad,), N, jnp.int32)])
  src_p = jnp.concatenate([src, jnp.full((pad,), N, jnp.int32)])
  offs = jnp.arange(B, dtype=jnp.int32) * N
  srcm = (src_g[None, :] + offs[:, None]).reshape(B, NS, NCH, CHUNK)
  dstm = dst_p.reshape(NS, NCH, CHUNK)
  src_as_dst = src_p.reshape(NS, NCH, CHUNK)

  zeros128 = jnp.zeros((NPAD, 128), jnp.float32)
  ones_blk = jnp.ones((CHUNK, 128), jnp.float32)

  deg_out = _sc_count(ones_blk, srcm, src_as_dst, zeros128)[0, :N]
  deg_in = _sc_count(ones_blk, srcm, dstm, zeros128)[0, :N]

  xs0 = pl.pallas_call(
      _prep_body,
      grid=_grid(),
      in_specs=[_row_spec(128), _deg_spec()],
      out_specs=_row_spec(128),
      out_shape=jax.ShapeDtypeStruct((B, N, 128), jnp.float32),
  )(node_feature, deg_out)

  agg0 = _sc_agg_128(xs0.reshape(B * N, 128), srcm, dstm, zeros128)[:, :N]

  h1, h1s_lo, h1s_hi = pl.pallas_call(
      _l0_body,
      grid=_grid(),
      in_specs=[_row_spec(128), _deg_spec(), _deg_spec(),
                _full_spec((128, 256)), _full_spec((1, 256))],
      out_specs=[_row_spec(256), _row_spec(128), _row_spec(128)],
      out_shape=[jax.ShapeDtypeStruct((B, N, 256), jnp.float32),
                 jax.ShapeDtypeStruct((B, N, 128), jnp.float32),
                 jax.ShapeDtypeStruct((B, N, 128), jnp.float32)],
  )(agg0, deg_in, deg_out, W0, b0.reshape(1, 256))

  agg1_lo = _sc_agg_128(h1s_lo.reshape(B * N, 128), srcm, dstm, zeros128)[:, :N]
  agg1_hi = _sc_agg_128(h1s_hi.reshape(B * N, 128), srcm, dstm, zeros128)[:, :N]

  h2, g2 = pl.pallas_call(
      _l1_body,
      grid=_grid(),
      in_specs=[_row_spec(128), _row_spec(128), _deg_spec(), _deg_spec(),
                _full_spec((256, 256)), _full_spec((1, 256)),
                _full_spec((256, 128))],
      out_specs=[_row_spec(256), _row_spec(128)],
      out_shape=[jax.ShapeDtypeStruct((B, N, 256), jnp.float32),
                 jax.ShapeDtypeStruct((B, N, 128), jnp.float32)],
  )(agg1_lo, agg1_hi, deg_in, deg_out, W1, b1.reshape(1, 256), W2)

  agg2 = _sc_agg_128(g2.reshape(B * N, 128), srcm, dstm, zeros128)[:, :N]

  y3 = pl.pallas_call(
      _l2_body,
      grid=_grid(),
      in_specs=[_row_spec(128), _deg_spec(), _full_spec((1, 128))],
      out_specs=_row_spec(128),
      out_shape=jax.ShapeDtypeStruct((B, N, 128), jnp.float32),
  )(agg2, deg_in, b2.reshape(1, 128))

  return jnp.concatenate([node_feature, h1, h2, y3], axis=-1)


# speedup vs baseline: 1.2913x; 1.2913x over previous
import functools

import jax
import jax.numpy as jnp
from jax import lax
from jax.experimental import pallas as pl
from jax.experimental.pallas import tpu as pltpu
from jax.experimental.pallas import tpu_sc as plsc

N = 10000
B = 2
E = 160000
NS = 16
CHUNK = 128
NCH = 80
IG = 8
NOG = NCH // IG
E_PAD = NS * NCH * CHUNK
RPT = 632
NPAD = NS * RPT
BLK = 1000


_MESH = plsc.VectorSubcoreMesh(core_axis_name="c", subcore_axis_name="s")

_AGG_SCRATCH = [
    pltpu.VMEM((IG, CHUNK), jnp.int32),
    pltpu.VMEM((IG, CHUNK), jnp.int32),
    pltpu.VMEM((2, CHUNK, 128), jnp.float32),
    pltpu.VMEM_SHARED((NPAD, 128), jnp.float32),
    pltpu.SemaphoreType.DMA,
    pltpu.SemaphoreType.DMA,
    pltpu.SemaphoreType.DMA,
    pltpu.SemaphoreType.DMA,
]


def _zero_accum(zeros, accum, s):
  pltpu.sync_copy(zeros.at[pl.ds(s * RPT, RPT)],
                  accum.at[pl.ds(s * RPT, RPT)])


def _copy_out(accum, out, c, s):
  pltpu.sync_copy(accum.at[pl.ds(s * RPT, RPT)],
                  out.at[c, pl.ds(s * RPT, RPT)])


def _agg_phase(tbl, srcm, dstm, c, s, src_i, dst_i, rows_v, accum,
               gsems, ssems):
  def body(og, carry):
    pltpu.sync_copy(srcm.at[c, s, pl.ds(og * IG, IG)], src_i)
    pltpu.sync_copy(dstm.at[s, pl.ds(og * IG, IG)], dst_i)
    gd = [None] * IG
    sd = [None] * IG
    gd[0] = pltpu.async_copy(tbl.at[src_i.at[0]], rows_v.at[0], gsems[0])
    for j in range(IG):
      if j >= 1:
        sd[j - 1].wait()
      if j + 1 < IG:
        gd[j + 1] = pltpu.async_copy(tbl.at[src_i.at[j + 1]],
                                     rows_v.at[(j + 1) % 2],
                                     gsems[(j + 1) % 2])
      gd[j].wait()
      sd[j] = pltpu.async_copy(rows_v.at[j % 2], accum.at[dst_i.at[j]],
                               ssems[j % 2], add=True)
    sd[IG - 1].wait()
    return carry

  lax.fori_loop(0, NOG, body, 0)


@functools.partial(
    pl.kernel,
    mesh=_MESH,
    out_type=jax.ShapeDtypeStruct((B, NPAD, 128), jnp.float32),
    scratch_types=_AGG_SCRATCH,
)
def _sc_agg_128(tbl, srcm, dstm, zeros, out, src_i, dst_i, rows_v, accum,
                gs0, gs1, ss0, ss1):
  c = lax.axis_index("c")
  s = lax.axis_index("s")
  _zero_accum(zeros, accum, s)
  plsc.subcore_barrier()
  _agg_phase(tbl, srcm, dstm, c, s, src_i, dst_i, rows_v, accum,
             [gs0, gs1], [ss0, ss1])
  plsc.subcore_barrier()
  _copy_out(accum, out, c, s)


@functools.partial(
    pl.kernel,
    mesh=_MESH,
    out_type=[jax.ShapeDtypeStruct((B, NPAD, 128), jnp.float32),
              jax.ShapeDtypeStruct((B, NPAD, 128), jnp.float32)],
    scratch_types=_AGG_SCRATCH,
)
def _sc_agg_128x2(tbl_lo, tbl_hi, srcm, dstm, zeros, out_lo, out_hi,
                  src_i, dst_i, rows_v, accum, gs0, gs1, ss0, ss1):
  c = lax.axis_index("c")
  s = lax.axis_index("s")
  for tbl, out in ((tbl_lo, out_lo), (tbl_hi, out_hi)):
    _zero_accum(zeros, accum, s)
    plsc.subcore_barrier()
    _agg_phase(tbl, srcm, dstm, c, s, src_i, dst_i, rows_v, accum,
               [gs0, gs1], [ss0, ss1])
    plsc.subcore_barrier()
    _copy_out(accum, out, c, s)
    plsc.subcore_barrier()


@functools.partial(
    pl.kernel,
    mesh=_MESH,
    out_type=[jax.ShapeDtypeStruct((B, NPAD, 128), jnp.float32),
              jax.ShapeDtypeStruct((B, NPAD, 128), jnp.float32)],
    scratch_types=_AGG_SCRATCH,
)
def _sc_count2(ones, srcm_h, dstm_h, zeros, out_o, out_i,
               src_i, dst_i, rows_v, accum, gs0, gs1, ss0, ss1):
  c = lax.axis_index("c")
  s = lax.axis_index("s")
  pltpu.sync_copy(ones, rows_v.at[0])
  for idxm, out in ((srcm_h, out_o), (dstm_h, out_i)):
    _zero_accum(zeros, accum, s)
    plsc.subcore_barrier()

    def body(og, carry):
      pltpu.sync_copy(idxm.at[s, pl.ds(og * IG, IG)], dst_i)
      for j in range(IG):
        pltpu.sync_copy(rows_v.at[0], accum.at[dst_i.at[j]], add=True)
      return carry

    lax.fori_loop(0, NOG, body, 0)
    plsc.subcore_barrier()
    _copy_out(accum, out, c, s)
    plsc.subcore_barrier()


def _dinv(deg_ref):
  return lax.rsqrt(jnp.maximum(deg_ref[:, 0:1], 1.0))


def _prep_body(x_ref, dout_ref, o_ref):
  o_ref[0] = x_ref[0] * _dinv(dout_ref)


def _l0_body(a_ref, din_ref, dout_ref, w_ref, b_ref, h_ref, lo_ref, hi_ref):
  a = a_ref[0] * _dinv(din_ref)
  h = jnp.dot(a, w_ref[...], preferred_element_type=jnp.float32) + b_ref[...]
  h = jnp.maximum(h, 0.0)
  h_ref[0] = h
  hs = h * _dinv(dout_ref)
  lo_ref[0] = hs[:, :128]
  hi_ref[0] = hs[:, 128:]


def _l1_body(alo_ref, ahi_ref, din_ref, dout_ref, w1_ref, b1_ref, w2_ref,
             h_ref, g_ref):
  din = _dinv(din_ref)
  h = (jnp.dot(alo_ref[0] * din, w1_ref[:128, :],
               preferred_element_type=jnp.float32)
       + jnp.dot(ahi_ref[0] * din, w1_ref[128:, :],
                 preferred_element_type=jnp.float32)
       + b1_ref[...])
  h = jnp.maximum(h, 0.0)
  h_ref[0] = h
  g_ref[0] = jnp.dot(h * _dinv(dout_ref), w2_ref[...],
                     preferred_element_type=jnp.float32)


def _l2_body(a_ref, din_ref, b_ref, o_ref):
  o_ref[0] = a_ref[0] * _dinv(din_ref) + b_ref[...]


def _row_spec(width):
  return pl.BlockSpec((1, BLK, width), lambda b, i: (b, i, 0))


def _deg_spec():
  return pl.BlockSpec((BLK, 128), lambda b, i: (i, 0))


def _full_spec(shape):
  return pl.BlockSpec(shape, lambda b, i: tuple(0 for _ in shape))


def _grid():
  return (B, N // BLK)


def kernel(node_feature, edge_index, W0, b0, W1, b1, W2, b2):
  src = edge_index[0].astype(jnp.int32)
  dst = edge_index[1].astype(jnp.int32)
  pad = E_PAD - E
  src_g = jnp.concatenate([src, jnp.zeros((pad,), jnp.int32)])
  dst_p = jnp.concatenate([dst, jnp.full((pad,), N, jnp.int32)])
  src_p = jnp.concatenate([src, jnp.full((pad,), N, jnp.int32)])
  offs = jnp.arange(B, dtype=jnp.int32) * N
  srcm = (src_g[None, :] + offs[:, None]).reshape(B, NS, NCH, CHUNK)
  dstm = dst_p.reshape(NS, NCH, CHUNK)
  src_as_dst = src_p.reshape(NS, NCH, CHUNK)

  zeros128 = jnp.zeros((NPAD, 128), jnp.float32)
  ones_blk = jnp.ones((CHUNK, 128), jnp.float32)

  deg_out, deg_in = _sc_count2(ones_blk, src_as_dst, dstm, zeros128)
  deg_out = deg_out[0, :N]
  deg_in = deg_in[0, :N]

  xs0 = pl.pallas_call(
      _prep_body,
      grid=_grid(),
      in_specs=[_row_spec(128), _deg_spec()],
      out_specs=_row_spec(128),
      out_shape=jax.ShapeDtypeStruct((B, N, 128), jnp.float32),
  )(node_feature, deg_out)

  agg0 = _sc_agg_128(xs0.reshape(B * N, 128), srcm, dstm, zeros128)[:, :N]

  h1, h1s_lo, h1s_hi = pl.pallas_call(
      _l0_body,
      grid=_grid(),
      in_specs=[_row_spec(128), _deg_spec(), _deg_spec(),
                _full_spec((128, 256)), _full_spec((1, 256))],
      out_specs=[_row_spec(256), _row_spec(128), _row_spec(128)],
      out_shape=[jax.ShapeDtypeStruct((B, N, 256), jnp.float32),
                 jax.ShapeDtypeStruct((B, N, 128), jnp.float32),
                 jax.ShapeDtypeStruct((B, N, 128), jnp.float32)],
  )(agg0, deg_in, deg_out, W0, b0.reshape(1, 256))

  agg1_lo, agg1_hi = _sc_agg_128x2(h1s_lo.reshape(B * N, 128),
                                   h1s_hi.reshape(B * N, 128),
                                   srcm, dstm, zeros128)
  agg1_lo = agg1_lo[:, :N]
  agg1_hi = agg1_hi[:, :N]

  h2, g2 = pl.pallas_call(
      _l1_body,
      grid=_grid(),
      in_specs=[_row_spec(128), _row_spec(128), _deg_spec(), _deg_spec(),
                _full_spec((256, 256)), _full_spec((1, 256)),
                _full_spec((256, 128))],
      out_specs=[_row_spec(256), _row_spec(128)],
      out_shape=[jax.ShapeDtypeStruct((B, N, 256), jnp.float32),
                 jax.ShapeDtypeStruct((B, N, 128), jnp.float32)],
  )(agg1_lo, agg1_hi, deg_in, deg_out, W1, b1.reshape(1, 256), W2)

  agg2 = _sc_agg_128(g2.reshape(B * N, 128), srcm, dstm, zeros128)[:, :N]

  y3 = pl.pallas_call(
      _l2_body,
      grid=_grid(),
      in_specs=[_row_spec(128), _deg_spec(), _full_spec((1, 128))],
      out_specs=_row_spec(128),
      out_shape=jax.ShapeDtypeStruct((B, N, 128), jnp.float32),
  )(agg2, deg_in, b2.reshape(1, 128))

  return jnp.concatenate([node_feature, h1, h2, y3], axis=-1)


# speedup vs baseline: 1.3293x; 1.0295x over previous
import functools

import jax
import jax.numpy as jnp
from jax import lax
from jax.experimental import pallas as pl
from jax.experimental.pallas import tpu as pltpu
from jax.experimental.pallas import tpu_sc as plsc

N = 10000
B = 2
E = 160000
NS = 16
CHUNK = 128
NCH = 80
IG = 16
NOG = NCH // IG
E_PAD = NS * NCH * CHUNK
RPT = 632
NPAD = NS * RPT
BLK = 1000


_MESH = plsc.VectorSubcoreMesh(core_axis_name="c", subcore_axis_name="s")

_AGG_SCRATCH = [
    pltpu.VMEM((IG, CHUNK), jnp.int32),
    pltpu.VMEM((IG, CHUNK), jnp.int32),
    pltpu.VMEM((2, CHUNK, 128), jnp.float32),
    pltpu.VMEM_SHARED((NPAD, 128), jnp.float32),
    pltpu.SemaphoreType.DMA,
    pltpu.SemaphoreType.DMA,
    pltpu.SemaphoreType.DMA,
    pltpu.SemaphoreType.DMA,
]


def _zero_accum(zeros, accum, s):
  pltpu.sync_copy(zeros.at[pl.ds(s * RPT, RPT)],
                  accum.at[pl.ds(s * RPT, RPT)])


def _copy_out(accum, out, c, s):
  pltpu.sync_copy(accum.at[pl.ds(s * RPT, RPT)],
                  out.at[c, pl.ds(s * RPT, RPT)])


def _agg_phase(tbl, srcm, dstm, c, s, src_i, dst_i, rows_v, accum,
               gsems, ssems):
  def body(og, carry):
    pltpu.sync_copy(srcm.at[c, s, pl.ds(og * IG, IG)], src_i)
    pltpu.sync_copy(dstm.at[s, pl.ds(og * IG, IG)], dst_i)
    gd = [None] * IG
    sd = [None] * IG
    gd[0] = pltpu.async_copy(tbl.at[src_i.at[0]], rows_v.at[0], gsems[0])
    for j in range(IG):
      if j >= 1:
        sd[j - 1].wait()
      if j + 1 < IG:
        gd[j + 1] = pltpu.async_copy(tbl.at[src_i.at[j + 1]],
                                     rows_v.at[(j + 1) % 2],
                                     gsems[(j + 1) % 2])
      gd[j].wait()
      sd[j] = pltpu.async_copy(rows_v.at[j % 2], accum.at[dst_i.at[j]],
                               ssems[j % 2], add=True)
    sd[IG - 1].wait()
    return carry

  lax.fori_loop(0, NOG, body, 0)


@functools.partial(
    pl.kernel,
    mesh=_MESH,
    out_type=jax.ShapeDtypeStruct((B, NPAD, 128), jnp.float32),
    scratch_types=_AGG_SCRATCH,
)
def _sc_agg_128(tbl, srcm, dstm, zeros, out, src_i, dst_i, rows_v, accum,
                gs0, gs1, ss0, ss1):
  c = lax.axis_index("c")
  s = lax.axis_index("s")
  _zero_accum(zeros, accum, s)
  plsc.subcore_barrier()
  _agg_phase(tbl, srcm, dstm, c, s, src_i, dst_i, rows_v, accum,
             [gs0, gs1], [ss0, ss1])
  plsc.subcore_barrier()
  _copy_out(accum, out, c, s)


@functools.partial(
    pl.kernel,
    mesh=_MESH,
    out_type=[jax.ShapeDtypeStruct((B, NPAD, 128), jnp.float32),
              jax.ShapeDtypeStruct((B, NPAD, 128), jnp.float32)],
    scratch_types=_AGG_SCRATCH,
)
def _sc_agg_128x2(tbl_lo, tbl_hi, srcm, dstm, zeros, out_lo, out_hi,
                  src_i, dst_i, rows_v, accum, gs0, gs1, ss0, ss1):
  c = lax.axis_index("c")
  s = lax.axis_index("s")
  for tbl, out in ((tbl_lo, out_lo), (tbl_hi, out_hi)):
    _zero_accum(zeros, accum, s)
    plsc.subcore_barrier()
    _agg_phase(tbl, srcm, dstm, c, s, src_i, dst_i, rows_v, accum,
               [gs0, gs1], [ss0, ss1])
    plsc.subcore_barrier()
    _copy_out(accum, out, c, s)
    plsc.subcore_barrier()


@functools.partial(
    pl.kernel,
    mesh=_MESH,
    out_type=[jax.ShapeDtypeStruct((B, NPAD, 128), jnp.float32),
              jax.ShapeDtypeStruct((B, NPAD, 128), jnp.float32)],
    scratch_types=_AGG_SCRATCH,
)
def _sc_count2(ones, srcm_h, dstm_h, zeros, out_o, out_i,
               src_i, dst_i, rows_v, accum, gs0, gs1, ss0, ss1):
  c = lax.axis_index("c")
  s = lax.axis_index("s")
  pltpu.sync_copy(ones, rows_v.at[0])
  for idxm, out in ((srcm_h, out_o), (dstm_h, out_i)):
    _zero_accum(zeros, accum, s)
    plsc.subcore_barrier()

    def body(og, carry):
      pltpu.sync_copy(idxm.at[s, pl.ds(og * IG, IG)], dst_i)
      for j in range(IG):
        pltpu.sync_copy(rows_v.at[0], accum.at[dst_i.at[j]], add=True)
      return carry

    lax.fori_loop(0, NOG, body, 0)
    plsc.subcore_barrier()
    _copy_out(accum, out, c, s)
    plsc.subcore_barrier()


def _dinv(deg_ref):
  return lax.rsqrt(jnp.maximum(deg_ref[:, 0:1], 1.0))


def _prep_body(x_ref, dout_ref, o_ref):
  o_ref[0] = x_ref[0] * _dinv(dout_ref)


def _l0_body(a_ref, din_ref, dout_ref, w_ref, b_ref, h_ref, lo_ref, hi_ref):
  a = a_ref[0] * _dinv(din_ref)
  h = jnp.dot(a, w_ref[...], preferred_element_type=jnp.float32) + b_ref[...]
  h = jnp.maximum(h, 0.0)
  h_ref[0] = h
  hs = h * _dinv(dout_ref)
  lo_ref[0] = hs[:, :128]
  hi_ref[0] = hs[:, 128:]


def _l1_body(alo_ref, ahi_ref, din_ref, dout_ref, w1_ref, b1_ref, w2_ref,
             h_ref, g_ref):
  din = _dinv(din_ref)
  h = (jnp.dot(alo_ref[0] * din, w1_ref[:128, :],
               preferred_element_type=jnp.float32)
       + jnp.dot(ahi_ref[0] * din, w1_ref[128:, :],
                 preferred_element_type=jnp.float32)
       + b1_ref[...])
  h = jnp.maximum(h, 0.0)
  h_ref[0] = h
  g_ref[0] = jnp.dot(h * _dinv(dout_ref), w2_ref[...],
                     preferred_element_type=jnp.float32)


def _l2_body(a_ref, din_ref, b_ref, o_ref):
  o_ref[0] = a_ref[0] * _dinv(din_ref) + b_ref[...]


def _row_spec(width):
  return pl.BlockSpec((1, BLK, width), lambda b, i: (b, i, 0))


def _deg_spec():
  return pl.BlockSpec((BLK, 128), lambda b, i: (i, 0))


def _full_spec(shape):
  return pl.BlockSpec(shape, lambda b, i: tuple(0 for _ in shape))


def _grid():
  return (B, N // BLK)


def kernel(node_feature, edge_index, W0, b0, W1, b1, W2, b2):
  src = edge_index[0].astype(jnp.int32)
  dst = edge_index[1].astype(jnp.int32)
  pad = E_PAD - E
  src_g = jnp.concatenate([src, jnp.zeros((pad,), jnp.int32)])
  dst_p = jnp.concatenate([dst, jnp.full((pad,), N, jnp.int32)])
  src_p = jnp.concatenate([src, jnp.full((pad,), N, jnp.int32)])
  offs = jnp.arange(B, dtype=jnp.int32) * N
  srcm = (src_g[None, :] + offs[:, None]).reshape(B, NS, NCH, CHUNK)
  dstm = dst_p.reshape(NS, NCH, CHUNK)
  src_as_dst = src_p.reshape(NS, NCH, CHUNK)

  zeros128 = jnp.zeros((NPAD, 128), jnp.float32)
  ones_blk = jnp.ones((CHUNK, 128), jnp.float32)

  deg_out, deg_in = _sc_count2(ones_blk, src_as_dst, dstm, zeros128)
  deg_out = deg_out[0, :N]
  deg_in = deg_in[0, :N]

  xs0 = pl.pallas_call(
      _prep_body,
      grid=_grid(),
      in_specs=[_row_spec(128), _deg_spec()],
      out_specs=_row_spec(128),
      out_shape=jax.ShapeDtypeStruct((B, N, 128), jnp.float32),
  )(node_feature, deg_out)

  agg0 = _sc_agg_128(xs0.reshape(B * N, 128), srcm, dstm, zeros128)[:, :N]

  h1, h1s_lo, h1s_hi = pl.pallas_call(
      _l0_body,
      grid=_grid(),
      in_specs=[_row_spec(128), _deg_spec(), _deg_spec(),
                _full_spec((128, 256)), _full_spec((1, 256))],
      out_specs=[_row_spec(256), _row_spec(128), _row_spec(128)],
      out_shape=[jax.ShapeDtypeStruct((B, N, 256), jnp.float32),
                 jax.ShapeDtypeStruct((B, N, 128), jnp.float32),
                 jax.ShapeDtypeStruct((B, N, 128), jnp.float32)],
  )(agg0, deg_in, deg_out, W0, b0.reshape(1, 256))

  agg1_lo, agg1_hi = _sc_agg_128x2(h1s_lo.reshape(B * N, 128),
                                   h1s_hi.reshape(B * N, 128),
                                   srcm, dstm, zeros128)
  agg1_lo = agg1_lo[:, :N]
  agg1_hi = agg1_hi[:, :N]

  h2, g2 = pl.pallas_call(
      _l1_body,
      grid=_grid(),
      in_specs=[_row_spec(128), _row_spec(128), _deg_spec(), _deg_spec(),
                _full_spec((256, 256)), _full_spec((1, 256)),
                _full_spec((256, 128))],
      out_specs=[_row_spec(256), _row_spec(128)],
      out_shape=[jax.ShapeDtypeStruct((B, N, 256), jnp.float32),
                 jax.ShapeDtypeStruct((B, N, 128), jnp.float32)],
  )(agg1_lo, agg1_hi, deg_in, deg_out, W1, b1.reshape(1, 256), W2)

  agg2 = _sc_agg_128(g2.reshape(B * N, 128), srcm, dstm, zeros128)[:, :N]

  y3 = pl.pallas_call(
      _l2_body,
      grid=_grid(),
      in_specs=[_row_spec(128), _deg_spec(), _full_spec((1, 128))],
      out_specs=_row_spec(128),
      out_shape=jax.ShapeDtypeStruct((B, N, 128), jnp.float32),
  )(agg2, deg_in, b2.reshape(1, 128))

  return jnp.concatenate([node_feature, h1, h2, y3], axis=-1)


# speedup vs baseline: 1.3570x; 1.0208x over previous
import functools

import jax
import jax.numpy as jnp
from jax import lax
from jax.experimental import pallas as pl
from jax.experimental.pallas import tpu as pltpu
from jax.experimental.pallas import tpu_sc as plsc

N = 10000
B = 2
E = 160000
NS = 16
CHUNK = 128
NCH = 80
IG = 40
NOG = NCH // IG
E_PAD = NS * NCH * CHUNK
RPT = 632
NPAD = NS * RPT
BLK = 1000


_MESH = plsc.VectorSubcoreMesh(core_axis_name="c", subcore_axis_name="s")

_AGG_SCRATCH = [
    pltpu.VMEM((IG, CHUNK), jnp.int32),
    pltpu.VMEM((IG, CHUNK), jnp.int32),
    pltpu.VMEM((2, CHUNK, 128), jnp.float32),
    pltpu.VMEM_SHARED((NPAD, 128), jnp.float32),
    pltpu.SemaphoreType.DMA,
    pltpu.SemaphoreType.DMA,
    pltpu.SemaphoreType.DMA,
    pltpu.SemaphoreType.DMA,
]


def _zero_accum(zeros, accum, s):
  pltpu.sync_copy(zeros.at[pl.ds(s * RPT, RPT)],
                  accum.at[pl.ds(s * RPT, RPT)])


def _copy_out(accum, out, c, s):
  pltpu.sync_copy(accum.at[pl.ds(s * RPT, RPT)],
                  out.at[c, pl.ds(s * RPT, RPT)])


def _agg_phase(tbl, srcm, dstm, c, s, src_i, dst_i, rows_v, accum,
               gsems, ssems):
  def body(og, carry):
    pltpu.sync_copy(srcm.at[c, s, pl.ds(og * IG, IG)], src_i)
    pltpu.sync_copy(dstm.at[s, pl.ds(og * IG, IG)], dst_i)
    gd = [None] * IG
    sd = [None] * IG
    gd[0] = pltpu.async_copy(tbl.at[src_i.at[0]], rows_v.at[0], gsems[0])
    for j in range(IG):
      if j >= 1:
        sd[j - 1].wait()
      if j + 1 < IG:
        gd[j + 1] = pltpu.async_copy(tbl.at[src_i.at[j + 1]],
                                     rows_v.at[(j + 1) % 2],
                                     gsems[(j + 1) % 2])
      gd[j].wait()
      sd[j] = pltpu.async_copy(rows_v.at[j % 2], accum.at[dst_i.at[j]],
                               ssems[j % 2], add=True)
    sd[IG - 1].wait()
    return carry

  lax.fori_loop(0, NOG, body, 0)


@functools.partial(
    pl.kernel,
    mesh=_MESH,
    out_type=jax.ShapeDtypeStruct((B, NPAD, 128), jnp.float32),
    scratch_types=_AGG_SCRATCH,
)
def _sc_agg_128(tbl, srcm, dstm, zeros, out, src_i, dst_i, rows_v, accum,
                gs0, gs1, ss0, ss1):
  c = lax.axis_index("c")
  s = lax.axis_index("s")
  _zero_accum(zeros, accum, s)
  plsc.subcore_barrier()
  _agg_phase(tbl, srcm, dstm, c, s, src_i, dst_i, rows_v, accum,
             [gs0, gs1], [ss0, ss1])
  plsc.subcore_barrier()
  _copy_out(accum, out, c, s)


@functools.partial(
    pl.kernel,
    mesh=_MESH,
    out_type=[jax.ShapeDtypeStruct((B, NPAD, 128), jnp.float32),
              jax.ShapeDtypeStruct((B, NPAD, 128), jnp.float32)],
    scratch_types=_AGG_SCRATCH,
)
def _sc_agg_128x2(tbl_lo, tbl_hi, srcm, dstm, zeros, out_lo, out_hi,
                  src_i, dst_i, rows_v, accum, gs0, gs1, ss0, ss1):
  c = lax.axis_index("c")
  s = lax.axis_index("s")
  for tbl, out in ((tbl_lo, out_lo), (tbl_hi, out_hi)):
    _zero_accum(zeros, accum, s)
    plsc.subcore_barrier()
    _agg_phase(tbl, srcm, dstm, c, s, src_i, dst_i, rows_v, accum,
               [gs0, gs1], [ss0, ss1])
    plsc.subcore_barrier()
    _copy_out(accum, out, c, s)
    plsc.subcore_barrier()


@functools.partial(
    pl.kernel,
    mesh=_MESH,
    out_type=[jax.ShapeDtypeStruct((B, NPAD, 128), jnp.float32),
              jax.ShapeDtypeStruct((B, NPAD, 128), jnp.float32)],
    scratch_types=_AGG_SCRATCH,
)
def _sc_count2(ones, srcm_h, dstm_h, zeros, out_o, out_i,
               src_i, dst_i, rows_v, accum, gs0, gs1, ss0, ss1):
  c = lax.axis_index("c")
  s = lax.axis_index("s")
  pltpu.sync_copy(ones, rows_v.at[0])
  for idxm, out in ((srcm_h, out_o), (dstm_h, out_i)):
    _zero_accum(zeros, accum, s)
    plsc.subcore_barrier()

    def body(og, carry):
      pltpu.sync_copy(idxm.at[s, pl.ds(og * IG, IG)], dst_i)
      for j in range(IG):
        pltpu.sync_copy(rows_v.at[0], accum.at[dst_i.at[j]], add=True)
      return carry

    lax.fori_loop(0, NOG, body, 0)
    plsc.subcore_barrier()
    _copy_out(accum, out, c, s)
    plsc.subcore_barrier()


def _dinv(deg_ref):
  return lax.rsqrt(jnp.maximum(deg_ref[:, 0:1], 1.0))


def _prep_body(x_ref, dout_ref, o_ref):
  o_ref[0] = x_ref[0] * _dinv(dout_ref)


def _l0_body(a_ref, din_ref, dout_ref, w_ref, b_ref, h_ref, lo_ref, hi_ref):
  a = a_ref[0] * _dinv(din_ref)
  h = jnp.dot(a, w_ref[...], preferred_element_type=jnp.float32) + b_ref[...]
  h = jnp.maximum(h, 0.0)
  h_ref[0] = h
  hs = h * _dinv(dout_ref)
  lo_ref[0] = hs[:, :128]
  hi_ref[0] = hs[:, 128:]


def _l1_body(alo_ref, ahi_ref, din_ref, dout_ref, w1_ref, b1_ref, w2_ref,
             h_ref, g_ref):
  din = _dinv(din_ref)
  h = (jnp.dot(alo_ref[0] * din, w1_ref[:128, :],
               preferred_element_type=jnp.float32)
       + jnp.dot(ahi_ref[0] * din, w1_ref[128:, :],
                 preferred_element_type=jnp.float32)
       + b1_ref[...])
  h = jnp.maximum(h, 0.0)
  h_ref[0] = h
  g_ref[0] = jnp.dot(h * _dinv(dout_ref), w2_ref[...],
                     preferred_element_type=jnp.float32)


def _l2_body(a_ref, din_ref, b_ref, o_ref):
  o_ref[0] = a_ref[0] * _dinv(din_ref) + b_ref[...]


def _row_spec(width):
  return pl.BlockSpec((1, BLK, width), lambda b, i: (b, i, 0))


def _deg_spec():
  return pl.BlockSpec((BLK, 128), lambda b, i: (i, 0))


def _full_spec(shape):
  return pl.BlockSpec(shape, lambda b, i: tuple(0 for _ in shape))


def _grid():
  return (B, N // BLK)


def kernel(node_feature, edge_index, W0, b0, W1, b1, W2, b2):
  src = edge_index[0].astype(jnp.int32)
  dst = edge_index[1].astype(jnp.int32)
  pad = E_PAD - E
  src_g = jnp.concatenate([src, jnp.zeros((pad,), jnp.int32)])
  dst_p = jnp.concatenate([dst, jnp.full((pad,), N, jnp.int32)])
  src_p = jnp.concatenate([src, jnp.full((pad,), N, jnp.int32)])
  offs = jnp.arange(B, dtype=jnp.int32) * N
  srcm = (src_g[None, :] + offs[:, None]).reshape(B, NS, NCH, CHUNK)
  dstm = dst_p.reshape(NS, NCH, CHUNK)
  src_as_dst = src_p.reshape(NS, NCH, CHUNK)

  zeros128 = jnp.zeros((NPAD, 128), jnp.float32)
  ones_blk = jnp.ones((CHUNK, 128), jnp.float32)

  deg_out, deg_in = _sc_count2(ones_blk, src_as_dst, dstm, zeros128)
  deg_out = deg_out[0, :N]
  deg_in = deg_in[0, :N]

  xs0 = pl.pallas_call(
      _prep_body,
      grid=_grid(),
      in_specs=[_row_spec(128), _deg_spec()],
      out_specs=_row_spec(128),
      out_shape=jax.ShapeDtypeStruct((B, N, 128), jnp.float32),
  )(node_feature, deg_out)

  agg0 = _sc_agg_128(xs0.reshape(B * N, 128), srcm, dstm, zeros128)[:, :N]

  h1, h1s_lo, h1s_hi = pl.pallas_call(
      _l0_body,
      grid=_grid(),
      in_specs=[_row_spec(128), _deg_spec(), _deg_spec(),
                _full_spec((128, 256)), _full_spec((1, 256))],
      out_specs=[_row_spec(256), _row_spec(128), _row_spec(128)],
      out_shape=[jax.ShapeDtypeStruct((B, N, 256), jnp.float32),
                 jax.ShapeDtypeStruct((B, N, 128), jnp.float32),
                 jax.ShapeDtypeStruct((B, N, 128), jnp.float32)],
  )(agg0, deg_in, deg_out, W0, b0.reshape(1, 256))

  agg1_lo, agg1_hi = _sc_agg_128x2(h1s_lo.reshape(B * N, 128),
                                   h1s_hi.reshape(B * N, 128),
                                   srcm, dstm, zeros128)
  agg1_lo = agg1_lo[:, :N]
  agg1_hi = agg1_hi[:, :N]

  h2, g2 = pl.pallas_call(
      _l1_body,
      grid=_grid(),
      in_specs=[_row_spec(128), _row_spec(128), _deg_spec(), _deg_spec(),
                _full_spec((256, 256)), _full_spec((1, 256)),
                _full_spec((256, 128))],
      out_specs=[_row_spec(256), _row_spec(128)],
      out_shape=[jax.ShapeDtypeStruct((B, N, 256), jnp.float32),
                 jax.ShapeDtypeStruct((B, N, 128), jnp.float32)],
  )(agg1_lo, agg1_hi, deg_in, deg_out, W1, b1.reshape(1, 256), W2)

  agg2 = _sc_agg_128(g2.reshape(B * N, 128), srcm, dstm, zeros128)[:, :N]

  y3 = pl.pallas_call(
      _l2_body,
      grid=_grid(),
      in_specs=[_row_spec(128), _deg_spec(), _full_spec((1, 128))],
      out_specs=_row_spec(128),
      out_shape=jax.ShapeDtypeStruct((B, N, 128), jnp.float32),
  )(agg2, deg_in, b2.reshape(1, 128))

  return jnp.concatenate([node_feature, h1, h2, y3], axis=-1)
